# Initial kernel scaffold; baseline (speedup 1.0000x reference)
import jax
import jax.numpy as jnp
from jax.experimental import pallas as pl


def kernel(x, edge_index, batch, params):
    raise NotImplementedError("write your pallas kernel here")


# speedup vs baseline: 10.4728x; 10.4728x over previous
import functools

import numpy as np
import jax
import jax.numpy as jnp
from jax import lax
from jax.experimental import pallas as pl
from jax.experimental.pallas import tpu as pltpu
from jax.experimental.pallas import tpu_sc as plsc

N = 10000
E = 320000
NF = 128
H1 = 128
H2 = 64
G = 20
MAXN = 500
ISQ = np.float32(1.0 / np.sqrt(1.0 + 1e-5))

NC, NS = 2, 16
NW = NC * NS
EPW = E // NW
CH = 80
NCHUNK = EPW // CH
N_PAD = 10240
RPT = N_PAD // NS
DW = 128

BN = 1000
NB = N // BN
MP = 512


@functools.cache
def _sc_mesh():
    return plsc.VectorSubcoreMesh(core_axis_name="c", subcore_axis_name="s",
                                  num_cores=NC, num_subcores=NS)


@functools.cache
def _make_deg():
    @functools.partial(
        pl.kernel,
        out_type=jax.ShapeDtypeStruct((NC * N_PAD, DW), jnp.float32),
        mesh=_sc_mesh(),
        scratch_types=[
            pltpu.VMEM((2, CH), jnp.int32),
            pltpu.VMEM((CH, DW), jnp.float32),
            pltpu.VMEM_SHARED((N_PAD, DW), jnp.float32),
        ],
    )
    def _deg(col_hbm, ones_hbm, zeros_hbm, out_hbm, colb, onesb, acc):
        cid = lax.axis_index("c")
        sid = lax.axis_index("s")
        w = cid * NS + sid
        tb = sid * RPT
        pltpu.sync_copy(ones_hbm, onesb)
        pltpu.sync_copy(zeros_hbm.at[pl.ds(tb, RPT)], acc.at[pl.ds(tb, RPT)])
        plsc.subcore_barrier()

        def body(i, carry):
            base = w * EPW + i * CH
            pltpu.sync_copy(col_hbm.at[pl.ds(base, CH)], colb.at[0])
            pltpu.sync_copy(onesb, acc.at[colb.at[0]], add=True)
            return carry

        lax.fori_loop(0, NCHUNK, body, 0)
        plsc.subcore_barrier()
        pltpu.sync_copy(acc.at[pl.ds(tb, RPT)],
                        out_hbm.at[pl.ds(cid * N_PAD + tb, RPT)])

    return _deg


@functools.cache
def _make_agg(F):
    @functools.partial(
        pl.kernel,
        out_type=jax.ShapeDtypeStruct((NC * N_PAD, F), jnp.float32),
        mesh=_sc_mesh(),
        scratch_types=[
            pltpu.VMEM((2, CH), jnp.int32),
            pltpu.VMEM((2, CH), jnp.int32),
            pltpu.VMEM((2, CH, F), jnp.float32),
            pltpu.VMEM_SHARED((N_PAD, F), jnp.float32),
            pltpu.SemaphoreType.DMA,
        ],
    )
    def agg(row_hbm, col_hbm, hs_hbm, zeros_hbm, out_hbm,
            rowb, colb, gbuf, acc, sem):
        cid = lax.axis_index("c")
        sid = lax.axis_index("s")
        w = cid * NS + sid
        tb = sid * RPT
        pltpu.sync_copy(zeros_hbm.at[pl.ds(tb, RPT)], acc.at[pl.ds(tb, RPT)])
        plsc.subcore_barrier()

        def body(i, carry):
            base = w * EPW + i * CH
            pltpu.sync_copy(row_hbm.at[pl.ds(base, CH)], rowb.at[0])
            pltpu.sync_copy(col_hbm.at[pl.ds(base, CH)], colb.at[0])
            pltpu.async_copy(hs_hbm.at[rowb.at[0]], gbuf.at[0], sem).wait()
            pltpu.sync_copy(gbuf.at[0], acc.at[colb.at[0]], add=True)
            return carry

        lax.fori_loop(0, NCHUNK, body, 0)
        plsc.subcore_barrier()
        pltpu.sync_copy(acc.at[pl.ds(tb, RPT)],
                        out_hbm.at[pl.ds(cid * N_PAD + tb, RPT)])

    return agg


def _k1_body(x_ref, w1_ref, degp_ref, hs1_ref, dinv_ref):
    deg = degp_ref[0] + degp_ref[1] + 1.0
    dinv = lax.rsqrt(deg)
    dinv_ref[...] = dinv
    h = jnp.dot(x_ref[...], w1_ref[...], preferred_element_type=jnp.float32)
    hs1_ref[...] = h * dinv[:, 0:1]


_k1 = pl.pallas_call(
    _k1_body,
    grid=(NB,),
    in_specs=[
        pl.BlockSpec((BN, NF), lambda i: (i, 0)),
        pl.BlockSpec((NF, H1), lambda i: (0, 0)),
        pl.BlockSpec((NC, BN, DW), lambda i: (0, i, 0)),
    ],
    out_specs=[
        pl.BlockSpec((BN, H1), lambda i: (i, 0)),
        pl.BlockSpec((BN, DW), lambda i: (i, 0)),
    ],
    out_shape=[
        jax.ShapeDtypeStruct((N, H1), jnp.float32),
        jax.ShapeDtypeStruct((N, DW), jnp.float32),
    ],
)


def _k2_body(aggp_ref, hs1_ref, x_ref, dinv_ref, b1_ref, g1_ref, bb1_ref,
             w2_ref, sw_ref, sb_ref, hs2_ref, res_ref):
    dinv1 = dinv_ref[...][:, 0:1]
    g = aggp_ref[0] + aggp_ref[1] + hs1_ref[...]
    y = g * dinv1 + b1_ref[...]
    y = y * (g1_ref[...] * ISQ) + bb1_ref[...]
    z1 = jax.nn.relu(jax.nn.relu(y) + x_ref[...])
    hs2 = jnp.dot(z1, w2_ref[...], preferred_element_type=jnp.float32) * dinv1
    hs2_ref[...] = jnp.concatenate(
        [hs2, jnp.zeros((BN, H1 - H2), jnp.float32)], axis=1)
    res_ref[...] = jnp.dot(z1, sw_ref[...],
                           preferred_element_type=jnp.float32) + sb_ref[...]


_k2 = pl.pallas_call(
    _k2_body,
    grid=(NB,),
    in_specs=[
        pl.BlockSpec((NC, BN, H1), lambda i: (0, i, 0)),
        pl.BlockSpec((BN, H1), lambda i: (i, 0)),
        pl.BlockSpec((BN, NF), lambda i: (i, 0)),
        pl.BlockSpec((BN, DW), lambda i: (i, 0)),
        pl.BlockSpec((1, H1), lambda i: (0, 0)),
        pl.BlockSpec((1, H1), lambda i: (0, 0)),
        pl.BlockSpec((1, H1), lambda i: (0, 0)),
        pl.BlockSpec((H1, H2), lambda i: (0, 0)),
        pl.BlockSpec((H1, H2), lambda i: (0, 0)),
        pl.BlockSpec((1, H2), lambda i: (0, 0)),
    ],
    out_specs=[
        pl.BlockSpec((BN, H1), lambda i: (i, 0)),
        pl.BlockSpec((BN, H2), lambda i: (i, 0)),
    ],
    out_shape=[
        jax.ShapeDtypeStruct((N, H1), jnp.float32),
        jax.ShapeDtypeStruct((N, H2), jnp.float32),
    ],
)


def _k3_body(aggp_ref, hs2_ref, res_ref, dinv_ref, b2_ref, g2_ref, bb2_ref,
             f1w_ref, f1b_ref, fg_ref, fb_ref, f2w_ref, f2b_ref,
             z_ref, xr_ref, zg_ref):
    dinv1 = dinv_ref[...][:, 0:1]
    g = (aggp_ref[0, :, :H2] + aggp_ref[1, :, :H2]
         + hs2_ref[...][:, :H2])
    y = g * dinv1 + b2_ref[...]
    y = y * (g2_ref[...] * ISQ) + bb2_ref[...]
    z2 = jax.nn.relu(jax.nn.relu(y) + res_ref[...])
    nrm = jnp.sqrt(jnp.sum(z2 * z2, axis=1, keepdims=True))
    z = z2 / jnp.maximum(nrm, 1e-12)
    z_ref[...] = z
    f = jax.nn.relu(jnp.dot(z, f1w_ref[...],
                            preferred_element_type=jnp.float32) + f1b_ref[...])
    f = f * (fg_ref[...] * ISQ) + fb_ref[...]
    xr_ref[...] = jnp.dot(f, f2w_ref[...],
                          preferred_element_type=jnp.float32) + f2b_ref[...]
    ri = lax.broadcasted_iota(jnp.int32, (BN, 1), 0)
    neg = jnp.float32(-jnp.inf)
    zg0 = jnp.max(jnp.where(ri < MAXN, z, neg), axis=0, keepdims=True)
    zg1 = jnp.max(jnp.where(ri >= MAXN, z, neg), axis=0, keepdims=True)
    zg_ref[0] = jnp.concatenate([zg0, zg1], axis=0)


_k3 = pl.pallas_call(
    _k3_body,
    grid=(NB,),
    in_specs=[
        pl.BlockSpec((NC, BN, H1), lambda i: (0, i, 0)),
        pl.BlockSpec((BN, H1), lambda i: (i, 0)),
        pl.BlockSpec((BN, H2), lambda i: (i, 0)),
        pl.BlockSpec((BN, DW), lambda i: (i, 0)),
        pl.BlockSpec((1, H2), lambda i: (0, 0)),
        pl.BlockSpec((1, H2), lambda i: (0, 0)),
        pl.BlockSpec((1, H2), lambda i: (0, 0)),
        pl.BlockSpec((H2, H1), lambda i: (0, 0)),
        pl.BlockSpec((1, H1), lambda i: (0, 0)),
        pl.BlockSpec((1, H1), lambda i: (0, 0)),
        pl.BlockSpec((1, H1), lambda i: (0, 0)),
        pl.BlockSpec((H1, NF), lambda i: (0, 0)),
        pl.BlockSpec((1, NF), lambda i: (0, 0)),
    ],
    out_specs=[
        pl.BlockSpec((BN, H2), lambda i: (i, 0)),
        pl.BlockSpec((BN, NF), lambda i: (i, 0)),
        pl.BlockSpec((1, 2, H2), lambda i: (i, 0, 0)),
    ],
    out_shape=[
        jax.ShapeDtypeStruct((N, H2), jnp.float32),
        jax.ShapeDtypeStruct((N, NF), jnp.float32),
        jax.ShapeDtypeStruct((NB, 2, H2), jnp.float32),
    ],
)


def _k4_body(zp_ref, w_ref, adj_ref):
    zb = zp_ref[0]
    t = jnp.dot(zb, w_ref[...], preferred_element_type=jnp.float32)
    logits = lax.dot_general(t, zb, (((1,), (1,)), ((), ())),
                             preferred_element_type=jnp.float32)
    s = jax.nn.sigmoid(logits)
    r = lax.broadcasted_iota(jnp.int32, (MP, MP), 0)
    c = lax.broadcasted_iota(jnp.int32, (MP, MP), 1)
    mask = (r != c) & (r < MAXN) & (c < MAXN)
    adj_ref[0] = jnp.where(mask, s, 0.0)


_k4 = pl.pallas_call(
    _k4_body,
    grid=(G,),
    in_specs=[
        pl.BlockSpec((1, MP, H2), lambda i: (i, 0, 0)),
        pl.BlockSpec((H2, H2), lambda i: (0, 0)),
    ],
    out_specs=pl.BlockSpec((1, MP, MP), lambda i: (i, 0, 0)),
    out_shape=jax.ShapeDtypeStruct((G, MP, MP), jnp.float32),
)


def _k5_body(zg_ref, w1_ref, b1_ref, w2_ref, b2_ref, out_ref):
    h = jax.nn.relu(jnp.dot(zg_ref[...], w1_ref[...],
                            preferred_element_type=jnp.float32) + b1_ref[...])
    out_ref[...] = jnp.dot(h, w2_ref[...],
                           preferred_element_type=jnp.float32) + b2_ref[...]


_k5 = pl.pallas_call(
    _k5_body,
    grid=(1,),
    in_specs=[
        pl.BlockSpec((G, H2), lambda i: (0, 0)),
        pl.BlockSpec((H2, H2), lambda i: (0, 0)),
        pl.BlockSpec((1, H2), lambda i: (0, 0)),
        pl.BlockSpec((H2, H2), lambda i: (0, 0)),
        pl.BlockSpec((1, H2), lambda i: (0, 0)),
    ],
    out_specs=pl.BlockSpec((G, H2), lambda i: (0, 0)),
    out_shape=jax.ShapeDtypeStruct((G, H2), jnp.float32),
)


def kernel(x, edge_index, batch, params):
    p = params
    row = edge_index[0]
    col = edge_index[1]
    ones_dw = jnp.ones((CH, DW), jnp.float32)
    zeros_dw = jnp.zeros((N_PAD, DW), jnp.float32)
    zeros_h1 = jnp.zeros((N_PAD, H1), jnp.float32)

    degp = _make_deg()(col, ones_dw, zeros_dw).reshape(NC, N_PAD, DW)
    hs1, dinv = _k1(x, p['conv1_W'], degp)
    aggp1 = _make_agg(H1)(row, col, hs1, zeros_h1).reshape(NC, N_PAD, H1)
    hs2, res = _k2(aggp1, hs1, x, dinv,
                   p['conv1_b'].reshape(1, H1), p['bn1_g'].reshape(1, H1),
                   p['bn1_b'].reshape(1, H1), p['conv2_W'],
                   p['short2_W'], p['short2_b'].reshape(1, H2))
    aggp2 = _make_agg(H1)(row, col, hs2, zeros_h1).reshape(NC, N_PAD, H1)
    z, x_recon, zg3 = _k3(aggp2, hs2, res, dinv,
                          p['conv2_b'].reshape(1, H2),
                          p['bn2_g'].reshape(1, H2),
                          p['bn2_b'].reshape(1, H2),
                          p['fd1_W'], p['fd1_b'].reshape(1, H1),
                          p['fd_bn_g'].reshape(1, H1),
                          p['fd_bn_b'].reshape(1, H1),
                          p['fd2_W'], p['fd2_b'].reshape(1, NF))
    z_g = zg3.reshape(G, H2)
    zp = jnp.pad(z.reshape(G, MAXN, H2), ((0, 0), (0, MP - MAXN), (0, 0)))
    adj = _k4(zp, p['ed_W'])[:, :MAXN, :MAXN]
    z_g_mlp = _k5(z_g, p['ph1_W'], p['ph1_b'].reshape(1, H2),
                  p['ph2_W'], p['ph2_b'].reshape(1, H2))
    return (z, x_recon, adj, z_g, z_g_mlp)


# speedup vs baseline: 14.7043x; 1.4041x over previous
import functools

import numpy as np
import jax
import jax.numpy as jnp
from jax import lax
from jax.experimental import pallas as pl
from jax.experimental.pallas import tpu as pltpu
from jax.experimental.pallas import tpu_sc as plsc

N = 10000
E = 320000
NF = 128
H1 = 128
H2 = 64
G = 20
MAXN = 500
ISQ = np.float32(1.0 / np.sqrt(1.0 + 1e-5))

NC, NS = 2, 16
NW = NC * NS
EPW = E // NW
CH = 80
NCHUNK = EPW // CH
N_PAD = 10240
RPT = N_PAD // NS
DW = 128

BN = 1000
NB = N // BN
MP = 512


@functools.cache
def _sc_mesh():
    return plsc.VectorSubcoreMesh(core_axis_name="c", subcore_axis_name="s",
                                  num_cores=NC, num_subcores=NS)


@functools.cache
def _make_deg():
    @functools.partial(
        pl.kernel,
        out_type=jax.ShapeDtypeStruct((NC * N_PAD, DW), jnp.float32),
        mesh=_sc_mesh(),
        scratch_types=[
            pltpu.VMEM((2, CH), jnp.int32),
            pltpu.VMEM((CH, DW), jnp.float32),
            pltpu.VMEM_SHARED((N_PAD, DW), jnp.float32),
        ],
    )
    def _deg(col_hbm, ones_hbm, zeros_hbm, out_hbm, colb, onesb, acc):
        cid = lax.axis_index("c")
        sid = lax.axis_index("s")
        w = cid * NS + sid
        tb = sid * RPT
        pltpu.sync_copy(ones_hbm, onesb)
        pltpu.sync_copy(zeros_hbm.at[pl.ds(tb, RPT)], acc.at[pl.ds(tb, RPT)])
        plsc.subcore_barrier()

        def body(i, carry):
            base = w * EPW + i * CH
            pltpu.sync_copy(col_hbm.at[pl.ds(base, CH)], colb.at[0])
            pltpu.sync_copy(onesb, acc.at[colb.at[0]], add=True)
            return carry

        lax.fori_loop(0, NCHUNK, body, 0)
        plsc.subcore_barrier()
        pltpu.sync_copy(acc.at[pl.ds(tb, RPT)],
                        out_hbm.at[pl.ds(cid * N_PAD + tb, RPT)])

    return _deg


@functools.cache
def _make_agg(F):
    @functools.partial(
        pl.kernel,
        out_type=jax.ShapeDtypeStruct((NC * N_PAD, F), jnp.float32),
        mesh=_sc_mesh(),
        scratch_types=[
            pltpu.VMEM((2, CH), jnp.int32),
            pltpu.VMEM((2, CH), jnp.int32),
            pltpu.VMEM((2, CH, F), jnp.float32),
            pltpu.VMEM_SHARED((N_PAD, F), jnp.float32),
            pltpu.SemaphoreType.DMA,
        ],
    )
    def agg(row_hbm, col_hbm, hs_hbm, zeros_hbm, out_hbm,
            rowb, colb, gbuf, acc, sem):
        cid = lax.axis_index("c")
        sid = lax.axis_index("s")
        w = cid * NS + sid
        tb = sid * RPT
        pltpu.sync_copy(zeros_hbm.at[pl.ds(tb, RPT)], acc.at[pl.ds(tb, RPT)])
        plsc.subcore_barrier()

        def start(i, b):
            base = w * EPW + i * CH
            pltpu.sync_copy(row_hbm.at[pl.ds(base, CH)], rowb.at[b])
            pltpu.sync_copy(col_hbm.at[pl.ds(base, CH)], colb.at[b])
            pltpu.async_copy(hs_hbm.at[rowb.at[b]], gbuf.at[b], sem)

        start(0, 0)

        def body(i, carry):
            b = lax.rem(i, 2)
            nb = lax.rem(i + 1, 2)

            @pl.when(i + 1 < NCHUNK)
            def _():
                start(i + 1, nb)

            pltpu.make_async_copy(
                hs_hbm.at[pl.ds(0, CH)], gbuf.at[0], sem).wait()
            pltpu.sync_copy(gbuf.at[b], acc.at[colb.at[b]], add=True)
            return carry

        lax.fori_loop(0, NCHUNK, body, 0)
        plsc.subcore_barrier()
        pltpu.sync_copy(acc.at[pl.ds(tb, RPT)],
                        out_hbm.at[pl.ds(cid * N_PAD + tb, RPT)])

    return agg


def _k0_body(x_ref, w1_ref, h_ref):
    h_ref[...] = jnp.dot(x_ref[...], w1_ref[...],
                         preferred_element_type=jnp.float32)


_k0 = pl.pallas_call(
    _k0_body,
    grid=(NB,),
    in_specs=[
        pl.BlockSpec((BN, NF), lambda i: (i, 0)),
        pl.BlockSpec((NF, H1), lambda i: (0, 0)),
    ],
    out_specs=pl.BlockSpec((BN, H1), lambda i: (i, 0)),
    out_shape=jax.ShapeDtypeStruct((N, H1), jnp.float32),
)


def _k1_body(h_ref, degp_ref, hs1_ref, dinv_ref):
    deg = degp_ref[0] + degp_ref[1] + 1.0
    dinv = lax.rsqrt(deg)
    dinv_ref[...] = dinv
    hs1_ref[...] = h_ref[...] * dinv[:, 0:1]


_k1 = pl.pallas_call(
    _k1_body,
    grid=(NB,),
    in_specs=[
        pl.BlockSpec((BN, H1), lambda i: (i, 0)),
        pl.BlockSpec((NC, BN, DW), lambda i: (0, i, 0)),
    ],
    out_specs=[
        pl.BlockSpec((BN, H1), lambda i: (i, 0)),
        pl.BlockSpec((BN, DW), lambda i: (i, 0)),
    ],
    out_shape=[
        jax.ShapeDtypeStruct((N, H1), jnp.float32),
        jax.ShapeDtypeStruct((N, DW), jnp.float32),
    ],
)


def _k2_body(aggp_ref, hs1_ref, x_ref, dinv_ref, b1_ref, g1_ref, bb1_ref,
             w2_ref, sw_ref, sb_ref, hs2_ref, res_ref):
    dinv1 = dinv_ref[...][:, 0:1]
    g = aggp_ref[0] + aggp_ref[1] + hs1_ref[...]
    y = g * dinv1 + b1_ref[...]
    y = y * (g1_ref[...] * ISQ) + bb1_ref[...]
    z1 = jax.nn.relu(jax.nn.relu(y) + x_ref[...])
    hs2 = jnp.dot(z1, w2_ref[...], preferred_element_type=jnp.float32) * dinv1
    hs2_ref[...] = jnp.concatenate(
        [hs2, jnp.zeros((BN, H1 - H2), jnp.float32)], axis=1)
    res_ref[...] = jnp.dot(z1, sw_ref[...],
                           preferred_element_type=jnp.float32) + sb_ref[...]


_k2 = pl.pallas_call(
    _k2_body,
    grid=(NB,),
    in_specs=[
        pl.BlockSpec((NC, BN, H1), lambda i: (0, i, 0)),
        pl.BlockSpec((BN, H1), lambda i: (i, 0)),
        pl.BlockSpec((BN, NF), lambda i: (i, 0)),
        pl.BlockSpec((BN, DW), lambda i: (i, 0)),
        pl.BlockSpec((1, H1), lambda i: (0, 0)),
        pl.BlockSpec((1, H1), lambda i: (0, 0)),
        pl.BlockSpec((1, H1), lambda i: (0, 0)),
        pl.BlockSpec((H1, H2), lambda i: (0, 0)),
        pl.BlockSpec((H1, H2), lambda i: (0, 0)),
        pl.BlockSpec((1, H2), lambda i: (0, 0)),
    ],
    out_specs=[
        pl.BlockSpec((BN, H1), lambda i: (i, 0)),
        pl.BlockSpec((BN, H2), lambda i: (i, 0)),
    ],
    out_shape=[
        jax.ShapeDtypeStruct((N, H1), jnp.float32),
        jax.ShapeDtypeStruct((N, H2), jnp.float32),
    ],
)


def _k3_body(aggp_ref, hs2_ref, res_ref, dinv_ref, b2_ref, g2_ref, bb2_ref,
             f1w_ref, f1b_ref, fg_ref, fb_ref, f2w_ref, f2b_ref,
             z_ref, xr_ref, zg_ref):
    dinv1 = dinv_ref[...][:, 0:1]
    g = (aggp_ref[0, :, :H2] + aggp_ref[1, :, :H2]
         + hs2_ref[...][:, :H2])
    y = g * dinv1 + b2_ref[...]
    y = y * (g2_ref[...] * ISQ) + bb2_ref[...]
    z2 = jax.nn.relu(jax.nn.relu(y) + res_ref[...])
    nrm = jnp.sqrt(jnp.sum(z2 * z2, axis=1, keepdims=True))
    z = z2 / jnp.maximum(nrm, 1e-12)
    z_ref[...] = z
    f = jax.nn.relu(jnp.dot(z, f1w_ref[...],
                            preferred_element_type=jnp.float32) + f1b_ref[...])
    f = f * (fg_ref[...] * ISQ) + fb_ref[...]
    xr_ref[...] = jnp.dot(f, f2w_ref[...],
                          preferred_element_type=jnp.float32) + f2b_ref[...]
    ri = lax.broadcasted_iota(jnp.int32, (BN, 1), 0)
    neg = jnp.float32(-jnp.inf)
    zg0 = jnp.max(jnp.where(ri < MAXN, z, neg), axis=0, keepdims=True)
    zg1 = jnp.max(jnp.where(ri >= MAXN, z, neg), axis=0, keepdims=True)
    zg_ref[0] = jnp.concatenate([zg0, zg1], axis=0)


_k3 = pl.pallas_call(
    _k3_body,
    grid=(NB,),
    in_specs=[
        pl.BlockSpec((NC, BN, H1), lambda i: (0, i, 0)),
        pl.BlockSpec((BN, H1), lambda i: (i, 0)),
        pl.BlockSpec((BN, H2), lambda i: (i, 0)),
        pl.BlockSpec((BN, DW), lambda i: (i, 0)),
        pl.BlockSpec((1, H2), lambda i: (0, 0)),
        pl.BlockSpec((1, H2), lambda i: (0, 0)),
        pl.BlockSpec((1, H2), lambda i: (0, 0)),
        pl.BlockSpec((H2, H1), lambda i: (0, 0)),
        pl.BlockSpec((1, H1), lambda i: (0, 0)),
        pl.BlockSpec((1, H1), lambda i: (0, 0)),
        pl.BlockSpec((1, H1), lambda i: (0, 0)),
        pl.BlockSpec((H1, NF), lambda i: (0, 0)),
        pl.BlockSpec((1, NF), lambda i: (0, 0)),
    ],
    out_specs=[
        pl.BlockSpec((BN, H2), lambda i: (i, 0)),
        pl.BlockSpec((BN, NF), lambda i: (i, 0)),
        pl.BlockSpec((1, 2, H2), lambda i: (i, 0, 0)),
    ],
    out_shape=[
        jax.ShapeDtypeStruct((N, H2), jnp.float32),
        jax.ShapeDtypeStruct((N, NF), jnp.float32),
        jax.ShapeDtypeStruct((NB, 2, H2), jnp.float32),
    ],
)


def _k4_body(zp_ref, w_ref, adj_ref):
    zb = zp_ref[0]
    t = jnp.dot(zb, w_ref[...], preferred_element_type=jnp.float32)
    logits = lax.dot_general(t, zb, (((1,), (1,)), ((), ())),
                             preferred_element_type=jnp.float32)
    s = jax.nn.sigmoid(logits)
    r = lax.broadcasted_iota(jnp.int32, (MP, MP), 0)
    c = lax.broadcasted_iota(jnp.int32, (MP, MP), 1)
    mask = (r != c) & (r < MAXN) & (c < MAXN)
    adj_ref[0] = jnp.where(mask, s, 0.0)


_k4 = pl.pallas_call(
    _k4_body,
    grid=(G,),
    in_specs=[
        pl.BlockSpec((1, MP, H2), lambda i: (i, 0, 0)),
        pl.BlockSpec((H2, H2), lambda i: (0, 0)),
    ],
    out_specs=pl.BlockSpec((1, MP, MP), lambda i: (i, 0, 0)),
    out_shape=jax.ShapeDtypeStruct((G, MP, MP), jnp.float32),
)


def _k5_body(zg_ref, w1_ref, b1_ref, w2_ref, b2_ref, out_ref):
    h = jax.nn.relu(jnp.dot(zg_ref[...], w1_ref[...],
                            preferred_element_type=jnp.float32) + b1_ref[...])
    out_ref[...] = jnp.dot(h, w2_ref[...],
                           preferred_element_type=jnp.float32) + b2_ref[...]


_k5 = pl.pallas_call(
    _k5_body,
    grid=(1,),
    in_specs=[
        pl.BlockSpec((G, H2), lambda i: (0, 0)),
        pl.BlockSpec((H2, H2), lambda i: (0, 0)),
        pl.BlockSpec((1, H2), lambda i: (0, 0)),
        pl.BlockSpec((H2, H2), lambda i: (0, 0)),
        pl.BlockSpec((1, H2), lambda i: (0, 0)),
    ],
    out_specs=pl.BlockSpec((G, H2), lambda i: (0, 0)),
    out_shape=jax.ShapeDtypeStruct((G, H2), jnp.float32),
)


def kernel(x, edge_index, batch, params):
    p = params
    row = edge_index[0]
    col = edge_index[1]
    ones_dw = jnp.ones((CH, DW), jnp.float32)
    zeros_dw = jnp.zeros((N_PAD, DW), jnp.float32)
    zeros_h1 = jnp.zeros((N_PAD, H1), jnp.float32)

    h1 = _k0(x, p['conv1_W'])
    degp = _make_deg()(col, ones_dw, zeros_dw).reshape(NC, N_PAD, DW)
    hs1, dinv = _k1(h1, degp)
    aggp1 = _make_agg(H1)(row, col, hs1, zeros_h1).reshape(NC, N_PAD, H1)
    hs2, res = _k2(aggp1, hs1, x, dinv,
                   p['conv1_b'].reshape(1, H1), p['bn1_g'].reshape(1, H1),
                   p['bn1_b'].reshape(1, H1), p['conv2_W'],
                   p['short2_W'], p['short2_b'].reshape(1, H2))
    aggp2 = _make_agg(H1)(row, col, hs2, zeros_h1).reshape(NC, N_PAD, H1)
    z, x_recon, zg3 = _k3(aggp2, hs2, res, dinv,
                          p['conv2_b'].reshape(1, H2),
                          p['bn2_g'].reshape(1, H2),
                          p['bn2_b'].reshape(1, H2),
                          p['fd1_W'], p['fd1_b'].reshape(1, H1),
                          p['fd_bn_g'].reshape(1, H1),
                          p['fd_bn_b'].reshape(1, H1),
                          p['fd2_W'], p['fd2_b'].reshape(1, NF))
    z_g = zg3.reshape(G, H2)
    zp = jnp.pad(z.reshape(G, MAXN, H2), ((0, 0), (0, MP - MAXN), (0, 0)))
    adj = _k4(zp, p['ed_W'])[:, :MAXN, :MAXN]
    z_g_mlp = _k5(z_g, p['ph1_W'], p['ph1_b'].reshape(1, H2),
                  p['ph2_W'], p['ph2_b'].reshape(1, H2))
    return (z, x_recon, adj, z_g, z_g_mlp)
